# baseline (speedup 1.0000x reference)
import jax
import jax.numpy as jnp
from jax import lax
from jax.experimental import pallas as pl
from jax.experimental.pallas import tpu as pltpu
from jax.experimental.pallas import tpu_sc as plsc

N = 100000
NC = 25000
E = 400000
C = 128
EPS = 1e-5

NCP = 25088
NP = 100352
EP = 401408
ACC_ROWS = 792
EACC_ROWS = 6288


def _segmax_body(x_hbm, cl_hbm, xp_hbm, acc, cbuf, idbuf, clbuf, rows, sem):
    c = lax.axis_index("c")
    s = lax.axis_index("s")
    w = s * 2 + c
    lo = w * 784

    neg = jnp.full((16,), -jnp.inf, jnp.float32)

    def init_r(r, carry):
        for k in range(8):
            acc[r, pl.ds(k * 16, 16)] = neg
        return carry

    lax.fori_loop(0, ACC_ROWS, init_r, 0)

    def chunk_body(ch, carry):
        base = ch * 2048
        pltpu.sync_copy(cl_hbm.at[pl.ds(base, 2048)], cbuf)

        def scan_g(g, cnt):
            cv = cbuf[pl.ds(g * 16, 16)]
            m = (cv >= lo) & (cv < lo + 784)
            ids = base + g * 16 + lax.iota(jnp.int32, 16)
            cs = plsc.cumsum(m.astype(jnp.int32))
            pos = cnt + cs - 1
            plsc.store_scatter(idbuf, [pos], ids, mask=m)
            plsc.store_scatter(clbuf, [pos], cv - lo, mask=m)
            return cnt + cs[15]

        cnt = lax.fori_loop(0, 128, scan_g, jnp.int32(0))

        zid = jnp.zeros((16,), jnp.int32)
        dcl = jnp.full((16,), 784, jnp.int32)
        for t in range(4):
            idbuf[pl.ds(cnt + t * 16, 16)] = zid
            clbuf[pl.ds(cnt + t * 16, 16)] = dcl
        nblk = (cnt + 63) // 64

        def blk(b, carry2):
            pltpu.async_copy(x_hbm.at[idbuf.at[pl.ds(b * 64, 64)]], rows, sem).wait()

            def grp(g, carry3):
                clv = clbuf[pl.ds(b * 64 + g * 16, 16)]
                for jj in range(16):
                    cl = clv[jj]
                    r = g * 16 + jj
                    for k in range(8):
                        sl = pl.ds(k * 16, 16)
                        acc[cl, sl] = jnp.maximum(acc[cl, sl], rows[r, sl])
                return carry3

            lax.fori_loop(0, 4, grp, 0)
            return carry2

        lax.fori_loop(0, nblk, blk, 0)
        return carry

    lax.fori_loop(0, NP // 2048, chunk_body, 0)

    def fix_r(r, carry):
        for k in range(8):
            sl = pl.ds(k * 16, 16)
            v = acc[r, sl]
            acc[r, sl] = jnp.where(v == -jnp.inf, jnp.zeros((16,), jnp.float32), v)
        return carry

    lax.fori_loop(0, 784, fix_r, 0)
    pltpu.sync_copy(acc.at[pl.ds(0, 784)], xp_hbm.at[pl.ds(lo, 784)])


def _sc_segmax(x, clp):
    mesh = plsc.VectorSubcoreMesh(
        core_axis_name="c", subcore_axis_name="s", num_cores=2, num_subcores=16
    )
    f = pl.kernel(
        _segmax_body,
        out_type=jax.ShapeDtypeStruct((NCP, C), jnp.float32),
        mesh=mesh,
        compiler_params=pltpu.CompilerParams(needs_layout_passes=False),
        scratch_types=[
            pltpu.VMEM((ACC_ROWS, C), jnp.float32),
            pltpu.VMEM((2048,), jnp.int32),
            pltpu.VMEM((2176,), jnp.int32),
            pltpu.VMEM((2176,), jnp.int32),
            pltpu.VMEM((64, C), jnp.float32),
            pltpu.SemaphoreType.DMA,
        ],
    )
    return f(x, clp)


def _edge_body(xs2_hbm, src_hbm, sel_hbm, dst_hbm, int_hbm, s2_hbm,
               acc, cgidx, cdst2, cint, rows, ibuf, sbuf, lbuf, dbuf, zbuf, sem):
    c = lax.axis_index("c")
    s = lax.axis_index("s")

    z = jnp.zeros((16,), jnp.float32)

    def zr(r, carry):
        for k in range(8):
            zbuf[r, pl.ds(k * 16, 16)] = z
        return carry

    lax.fori_loop(0, 128, zr, 0)

    ebase = s * 25088

    for half in range(2):
        lo = c * 12544 + half * 6272

        zlo = s * 393
        for tz in range(3):
            pltpu.sync_copy(zbuf, acc.at[pl.ds(zlo + tz * 128, 128)])
        pltpu.sync_copy(zbuf.at[pl.ds(0, 9)], acc.at[pl.ds(zlo + 384, 9)])
        plsc.subcore_barrier()

        def chunk(chk, carry):
            cb = ebase + chk * 1792
            pltpu.sync_copy(int_hbm.at[pl.ds(cb, 1792)], ibuf)
            pltpu.sync_copy(src_hbm.at[pl.ds(cb, 1792)], sbuf)
            pltpu.sync_copy(sel_hbm.at[pl.ds(cb, 1792)], lbuf)
            pltpu.sync_copy(dst_hbm.at[pl.ds(cb, 1792)], dbuf)

            def scan_g(g, cnt):
                dv = dbuf[pl.ds(g * 16, 16)]
                m = (dv >= lo) & (dv < lo + 6272)
                sv = sbuf[pl.ds(g * 16, 16)]
                lv = lbuf[pl.ds(g * 16, 16)]
                iv = ibuf[pl.ds(g * 16, 16)]
                gi = lv * NCP + sv
                cs = plsc.cumsum(m.astype(jnp.int32))
                pos = cnt + cs - 1
                plsc.store_scatter(cgidx, [pos], gi, mask=m)
                plsc.store_scatter(cint, [pos], iv, mask=m)
                plsc.store_scatter(
                    cdst2, [pos // 128, pos % 128], dv - lo, mask=m
                )
                return cnt + cs[15]

            cnt = lax.fori_loop(0, 112, scan_g, jnp.int32(0))

            zgi = jnp.zeros((16,), jnp.int32)
            zin = jnp.zeros((16,), jnp.float32)
            dloc = jnp.full((16,), 6272, jnp.int32)
            io16 = lax.iota(jnp.int32, 16)
            for tp in range(8):
                pp = cnt + tp * 16 + io16
                plsc.store_scatter(cgidx, [pp], zgi)
                plsc.store_scatter(cint, [pp], zin)
                plsc.store_scatter(cdst2, [pp // 128, pp % 128], dloc)
            nblk = (cnt + 127) // 128

            def blk(j, carry2):
                pltpu.async_copy(
                    xs2_hbm.at[cgidx.at[pl.ds(j * 128, 128)]], rows, sem
                ).wait()

                def grp(g, carry3):
                    iv = cint[pl.ds(j * 128 + g * 16, 16)]
                    for jj in range(16):
                        sc = iv[jj]
                        r = g * 16 + jj
                        for k in range(8):
                            sl = pl.ds(k * 16, 16)
                            rows[r, sl] = rows[r, sl] * sc
                    return carry3

                lax.fori_loop(0, 8, grp, 0)
                pltpu.sync_copy(rows, acc.at[cdst2.at[j]], add=True)
                return carry2

            lax.fori_loop(0, nblk, blk, 0)
            return carry

        lax.fori_loop(0, 14, chunk, 0)
        plsc.subcore_barrier()
        pltpu.sync_copy(
            acc.at[pl.ds(s * 392, 392)],
            s2_hbm.at[pl.ds(lo + s * 392, 392)],
        )
        plsc.subcore_barrier()


def _sc_edge(xs2, srcp, selp, dstp, intp):
    mesh = plsc.VectorSubcoreMesh(
        core_axis_name="c", subcore_axis_name="s", num_cores=2, num_subcores=16
    )
    f = pl.kernel(
        _edge_body,
        out_type=jax.ShapeDtypeStruct((NCP, C), jnp.float32),
        mesh=mesh,
        compiler_params=pltpu.CompilerParams(needs_layout_passes=False),
        scratch_types=[
            pltpu.VMEM_SHARED((EACC_ROWS, C), jnp.float32),
            pltpu.VMEM((2048,), jnp.int32),
            pltpu.VMEM((16, 128), jnp.int32),
            pltpu.VMEM((2048,), jnp.float32),
            pltpu.VMEM((128, C), jnp.float32),
            pltpu.VMEM((1792,), jnp.float32),
            pltpu.VMEM((1792,), jnp.int32),
            pltpu.VMEM((1792,), jnp.int32),
            pltpu.VMEM((1792,), jnp.int32),
            pltpu.VMEM((128, C), jnp.float32),
            pltpu.SemaphoreType.DMA,
        ],
    )
    return f(xs2, srcp, selp, dstp, intp)


def _tc_pre_body(xp_ref, w1_ref, wr_ref, u1_ref, ur_ref, st_ref):
    i = pl.program_id(0)
    xb = xp_ref[...]
    u1 = jnp.dot(xb, w1_ref[...], preferred_element_type=jnp.float32)
    ur = jnp.dot(xb, wr_ref[...], preferred_element_type=jnp.float32)
    u1_ref[...] = u1
    ur_ref[...] = ur
    st = jnp.stack(
        [u1.sum(0), (u1 * u1).sum(0), ur.sum(0), (ur * ur).sum(0)]
    )

    @pl.when(i == 0)
    def _():
        st_ref[...] = st

    @pl.when(i > 0)
    def _():
        st_ref[...] = st_ref[...] + st


def _tc_pre(xp, W1, Wr):
    return pl.pallas_call(
        _tc_pre_body,
        grid=(32,),
        in_specs=[
            pl.BlockSpec((784, C), lambda i: (i, 0)),
            pl.BlockSpec((C, C), lambda i: (0, 0)),
            pl.BlockSpec((C, C), lambda i: (0, 0)),
        ],
        out_specs=[
            pl.BlockSpec((784, C), lambda i: (i, 0)),
            pl.BlockSpec((784, C), lambda i: (i, 0)),
            pl.BlockSpec((4, C), lambda i: (0, 0)),
        ],
        out_shape=[
            jax.ShapeDtypeStruct((NCP, C), jnp.float32),
            jax.ShapeDtypeStruct((NCP, C), jnp.float32),
            jax.ShapeDtypeStruct((4, C), jnp.float32),
        ],
    )(xp, W1, Wr)


def _tc_mid_body(u1_ref, st_ref, g1_ref, be1_ref, w2_ref, xs_ref):
    mu = st_ref[0, :] * (1.0 / NC)
    var = st_ref[1, :] * (1.0 / NC) - mu * mu
    inv = lax.rsqrt(var + EPS)
    h = jnp.maximum(
        (u1_ref[...] - mu[None, :]) * (inv * g1_ref[0, :])[None, :]
        + be1_ref[0, :][None, :],
        0.0,
    )
    for s9 in range(9):
        xs_ref[s9] = jnp.dot(h, w2_ref[s9], preferred_element_type=jnp.float32)


def _tc_mid(U1, st, g1, be1, W2):
    return pl.pallas_call(
        _tc_mid_body,
        grid=(32,),
        in_specs=[
            pl.BlockSpec((784, C), lambda i: (i, 0)),
            pl.BlockSpec((4, C), lambda i: (0, 0)),
            pl.BlockSpec((1, C), lambda i: (0, 0)),
            pl.BlockSpec((1, C), lambda i: (0, 0)),
            pl.BlockSpec((9, C, C), lambda i: (0, 0, 0)),
        ],
        out_specs=pl.BlockSpec((9, 784, C), lambda i: (0, i, 0)),
        out_shape=jax.ShapeDtypeStruct((9, NCP, C), jnp.float32),
    )(U1, st, g1.reshape(1, C), be1.reshape(1, C), W2)


def _tc_stats_body(s2_ref, st_ref):
    i = pl.program_id(0)
    xb = s2_ref[...]
    st = jnp.stack([xb.sum(0), (xb * xb).sum(0)])

    @pl.when(i == 0)
    def _():
        st_ref[...] = st

    @pl.when(i > 0)
    def _():
        st_ref[...] = st_ref[...] + st


def _tc_stats(S2):
    return pl.pallas_call(
        _tc_stats_body,
        grid=(32,),
        in_specs=[pl.BlockSpec((784, C), lambda i: (i, 0))],
        out_specs=pl.BlockSpec((2, C), lambda i: (0, 0)),
        out_shape=jax.ShapeDtypeStruct((2, C), jnp.float32),
    )(S2)


def _tc_post_body(s2_ref, st2_ref, g2_ref, be2_ref, w3_ref, b3_ref,
                  ur_ref, str_ref, gr_ref, ber_ref, y_ref):
    mu2 = st2_ref[0, :] * (1.0 / NC)
    var2 = st2_ref[1, :] * (1.0 / NC) - mu2 * mu2
    inv2 = lax.rsqrt(var2 + EPS)
    s2 = s2_ref[...]
    o3 = jnp.maximum(
        (s2 - mu2[None, :]) * (inv2 * g2_ref[0, :])[None, :]
        + be2_ref[0, :][None, :],
        0.0,
    )
    out4 = jnp.dot(o3, w3_ref[...], preferred_element_type=jnp.float32)
    out4 = out4 + b3_ref[0, :][None, :]
    mur = str_ref[2, :] * (1.0 / NC)
    varr = str_ref[3, :] * (1.0 / NC) - mur * mur
    invr = lax.rsqrt(varr + EPS)
    res = (ur_ref[...] - mur[None, :]) * (invr * gr_ref[0, :])[None, :] \
        + ber_ref[0, :][None, :]
    y_ref[...] = jnp.maximum(out4 + res, 0.0)


def _tc_post(S2, st2, g2, be2, W3, b3, Ur, st, gr, ber):
    return pl.pallas_call(
        _tc_post_body,
        grid=(25,),
        in_specs=[
            pl.BlockSpec((1000, C), lambda i: (i, 0)),
            pl.BlockSpec((2, C), lambda i: (0, 0)),
            pl.BlockSpec((1, C), lambda i: (0, 0)),
            pl.BlockSpec((1, C), lambda i: (0, 0)),
            pl.BlockSpec((C, C), lambda i: (0, 0)),
            pl.BlockSpec((1, C), lambda i: (0, 0)),
            pl.BlockSpec((1000, C), lambda i: (i, 0)),
            pl.BlockSpec((4, C), lambda i: (0, 0)),
            pl.BlockSpec((1, C), lambda i: (0, 0)),
            pl.BlockSpec((1, C), lambda i: (0, 0)),
        ],
        out_specs=pl.BlockSpec((1000, C), lambda i: (i, 0)),
        out_shape=jax.ShapeDtypeStruct((NC, C), jnp.float32),
    )(S2, st2, g2.reshape(1, C), be2.reshape(1, C), W3, b3.reshape(1, C),
      Ur, st, gr.reshape(1, C), ber.reshape(1, C))


def kernel(x, cluster, down_edge_index, down_selections, down_interps,
           W1, b1, g1, be1, W2, b2, g2, be2, W3, b3, Wr, br, gr, ber):
    del b1, b2, br

    clp = jnp.concatenate(
        [cluster, jnp.full((NP - N,), NCP, jnp.int32)]
    )
    xp = _sc_segmax(x, clp)

    U1, Ur, st = _tc_pre(xp, W1, Wr)
    xs = _tc_mid(U1, st, g1, be1, W2)

    src = down_edge_index[0]
    dst = down_edge_index[1]
    pad = EP - E
    srcp = jnp.concatenate([src, jnp.zeros((pad,), jnp.int32)])
    dstp = jnp.concatenate([dst, jnp.full((pad,), NCP, jnp.int32)])
    selp = jnp.concatenate([down_selections, jnp.zeros((pad,), jnp.int32)])
    intp = jnp.concatenate([down_interps, jnp.zeros((pad,), jnp.float32)])

    xs2 = xs.reshape(9 * NCP, C)
    S2 = _sc_edge(xs2, srcp, selp, dstp, intp)

    st2 = _tc_stats(S2)
    return _tc_post(S2, st2, g2, be2, W3, b3, Ur, st, gr, ber)


# speedup vs baseline: 1.4604x; 1.4604x over previous
import jax
import jax.numpy as jnp
from jax import lax
from jax.experimental import pallas as pl
from jax.experimental.pallas import tpu as pltpu
from jax.experimental.pallas import tpu_sc as plsc

N = 100000
NC = 25000
E = 400000
C = 128
EPS = 1e-5

NCP = 25088
NP = 100352
EP = 401408
EACC_ROWS = 3152
NCHUNKS = NP // 2048


def _segmax_body(x_hbm, cl_hbm, xp_hbm, acc, cbuf, idbuf, clbuf, rows,
                 sem, msem0, msem1):
    c = lax.axis_index("c")
    s = lax.axis_index("s")
    w = s * 2 + c
    lo = w * 784

    neg = jnp.full((16,), -jnp.inf, jnp.float32)

    def init_r(r, carry):
        for k in range(8):
            acc[r, pl.ds(k * 16, 16)] = neg
        return carry

    lax.fori_loop(0, 785, init_r, 0)

    def fire_meta(ch, msem):
        pltpu.async_copy(
            cl_hbm.at[pl.ds(ch * 2048, 2048)],
            cbuf.at[pl.ds((ch % 2) * 2048, 2048)],
            msem,
        )

    def wait_meta(ch, msem):
        pltpu.make_async_copy(
            cl_hbm.at[pl.ds(ch * 2048, 2048)],
            cbuf.at[pl.ds((ch % 2) * 2048, 2048)],
            msem,
        ).wait()

    fire_meta(jnp.int32(0), msem0)

    def rmw_block(boff):
        pltpu.async_copy(
            x_hbm.at[idbuf.at[pl.ds(boff, 128)]], rows, sem
        ).wait()

        def grp(g, carry3):
            clv = clbuf[pl.ds(boff + g * 16, 16)]
            for jj in range(16):
                cl = clv[jj]
                r = g * 16 + jj
                for k in range(8):
                    sl = pl.ds(k * 16, 16)
                    acc[cl, sl] = jnp.maximum(acc[cl, sl], rows[r, sl])
            return carry3

        lax.fori_loop(0, 8, grp, 0)

    def chunk_body(ch, cnt):
        base = ch * 2048
        cboff = (ch % 2) * 2048

        @pl.when(ch % 2 == 0)
        def _():
            wait_meta(ch, msem0)

        @pl.when(ch % 2 == 1)
        def _():
            wait_meta(ch, msem1)

        @pl.when((ch + 1 < NCHUNKS) & ((ch + 1) % 2 == 0))
        def _():
            fire_meta(ch + 1, msem0)

        @pl.when((ch + 1 < NCHUNKS) & ((ch + 1) % 2 == 1))
        def _():
            fire_meta(ch + 1, msem1)

        def scan_g(g, cnt2):
            cv = cbuf[pl.ds(cboff + g * 16, 16)]
            m = (cv >= lo) & (cv < lo + 784)
            ids = base + g * 16 + lax.iota(jnp.int32, 16)
            cs = plsc.cumsum(m.astype(jnp.int32))
            pos = cnt2 + cs - 1
            plsc.store_scatter(idbuf, [pos], ids, mask=m)
            plsc.store_scatter(clbuf, [pos], cv - lo, mask=m)
            return cnt2 + cs[15]

        cnt = lax.fori_loop(0, 128, scan_g, cnt)

        nblk = cnt // 128

        def blk(b, carry2):
            rmw_block(b * 128)
            return carry2

        lax.fori_loop(0, nblk, blk, 0)

        def mv(g, carry2):
            idbuf[pl.ds(g * 16, 16)] = idbuf[pl.ds(nblk * 128 + g * 16, 16)]
            clbuf[pl.ds(g * 16, 16)] = clbuf[pl.ds(nblk * 128 + g * 16, 16)]
            return carry2

        @pl.when(nblk > 0)
        def _():
            lax.fori_loop(0, 8, mv, 0)

        return cnt - nblk * 128

    cnt = lax.fori_loop(0, NCHUNKS, chunk_body, jnp.int32(0))

    zid = jnp.zeros((16,), jnp.int32)
    dcl = jnp.full((16,), 784, jnp.int32)
    for tp in range(8):
        idbuf[pl.ds(cnt + tp * 16, 16)] = zid
        clbuf[pl.ds(cnt + tp * 16, 16)] = dcl

    @pl.when(cnt > 0)
    def _():
        rmw_block(0)

    def fix_r(r, carry):
        for k in range(8):
            sl = pl.ds(k * 16, 16)
            v = acc[r, sl]
            acc[r, sl] = jnp.where(
                v == -jnp.inf, jnp.zeros((16,), jnp.float32), v
            )
        return carry

    lax.fori_loop(0, 784, fix_r, 0)
    pltpu.sync_copy(acc.at[pl.ds(0, 784)], xp_hbm.at[pl.ds(lo, 784)])


def _sc_segmax(x, clp):
    mesh = plsc.VectorSubcoreMesh(
        core_axis_name="c", subcore_axis_name="s", num_cores=2, num_subcores=16
    )
    f = pl.kernel(
        _segmax_body,
        out_type=jax.ShapeDtypeStruct((NCP, C), jnp.float32),
        mesh=mesh,
        compiler_params=pltpu.CompilerParams(needs_layout_passes=False),
        scratch_types=[
            pltpu.VMEM((785, C), jnp.float32),
            pltpu.VMEM((4096,), jnp.int32),
            pltpu.VMEM((2432,), jnp.int32),
            pltpu.VMEM((2432,), jnp.int32),
            pltpu.VMEM((128, C), jnp.float32),
            pltpu.SemaphoreType.DMA,
            pltpu.SemaphoreType.DMA,
            pltpu.SemaphoreType.DMA,
        ],
    )
    return f(x, clp)


def _edge_body(xs2_hbm, src_hbm, sel_hbm, dst_hbm, int_hbm, s2_hbm,
               acc, cgidx, cdst2, cint, rows, ibuf, sbuf, lbuf, dbuf, zbuf,
               gsem, ssem):
    c = lax.axis_index("c")
    s = lax.axis_index("s")

    z = jnp.zeros((16,), jnp.float32)

    def zr(r, carry):
        for k in range(8):
            zbuf[r, pl.ds(k * 16, 16)] = z
        return carry

    lax.fori_loop(0, 64, zr, 0)

    ebase = s * 25088

    for quarter in range(4):
        lo = c * 12544 + quarter * 3136

        zlo = s * 197
        for tz in range(3):
            pltpu.sync_copy(zbuf, acc.at[pl.ds(zlo + tz * 64, 64)])
        pltpu.sync_copy(zbuf.at[pl.ds(0, 5)], acc.at[pl.ds(zlo + 192, 5)])
        plsc.subcore_barrier()

        def chunk(chk, carry):
            cb = ebase + chk * 1792
            pltpu.sync_copy(
                (int_hbm.at[pl.ds(cb, 1792)], src_hbm.at[pl.ds(cb, 1792)],
                 sel_hbm.at[pl.ds(cb, 1792)], dst_hbm.at[pl.ds(cb, 1792)]),
                (ibuf, sbuf, lbuf, dbuf),
            )

            def scan_g(g, cnt):
                dv = dbuf[pl.ds(g * 16, 16)]
                m = (dv >= lo) & (dv < lo + 3136)
                sv = sbuf[pl.ds(g * 16, 16)]
                lv = lbuf[pl.ds(g * 16, 16)]
                iv = ibuf[pl.ds(g * 16, 16)]
                gi = lv * NCP + sv
                cs = plsc.cumsum(m.astype(jnp.int32))
                pos = cnt + cs - 1
                plsc.store_scatter(cgidx, [pos], gi, mask=m)
                plsc.store_scatter(cint, [pos], iv, mask=m)
                plsc.store_scatter(
                    cdst2, [pos // 128, pos % 128], dv - lo, mask=m
                )
                return cnt + cs[15]

            cnt = lax.fori_loop(0, 112, scan_g, jnp.int32(0))

            zgi = jnp.zeros((16,), jnp.int32)
            zin = jnp.zeros((16,), jnp.float32)
            dloc = jnp.full((16,), 3136, jnp.int32)
            io16 = lax.iota(jnp.int32, 16)
            for tp in range(8):
                pp = cnt + tp * 16 + io16
                plsc.store_scatter(cgidx, [pp], zgi)
                plsc.store_scatter(cint, [pp], zin)
                plsc.store_scatter(cdst2, [pp // 128, pp % 128], dloc)
            nblk = (cnt + 127) // 128

            def bat(bt, carry2):
                nb = jnp.minimum(nblk - bt * 4, 4)
                for q in range(4):
                    @pl.when(q < nb)
                    def _(q=q):
                        pltpu.async_copy(
                            xs2_hbm.at[
                                cgidx.at[pl.ds((bt * 4 + q) * 128, 128)]
                            ],
                            rows.at[pl.ds(q * 128, 128)],
                            gsem,
                        )
                for q in range(4):
                    @pl.when(q < nb)
                    def _(q=q):
                        pltpu.make_async_copy(
                            xs2_hbm.at[
                                cgidx.at[pl.ds((bt * 4 + q) * 128, 128)]
                            ],
                            rows.at[pl.ds(q * 128, 128)],
                            gsem,
                        ).wait()

                def grp(g, carry3):
                    iv = cint[pl.ds(bt * 512 + g * 16, 16)]
                    for jj in range(16):
                        sc = iv[jj]
                        r = g * 16 + jj
                        for k in range(8):
                            sl = pl.ds(k * 16, 16)
                            rows[r, sl] = rows[r, sl] * sc
                    return carry3

                lax.fori_loop(0, nb * 8, grp, 0)

                for q in range(4):
                    @pl.when(q < nb)
                    def _(q=q):
                        pltpu.make_async_copy(
                            rows.at[pl.ds(q * 128, 128)],
                            acc.at[cdst2.at[bt * 4 + q]],
                            ssem,
                        ).start(add=True)
                for q in range(4):
                    @pl.when(q < nb)
                    def _(q=q):
                        pltpu.make_async_copy(
                            rows.at[pl.ds(q * 128, 128)],
                            acc.at[cdst2.at[bt * 4 + q]],
                            ssem,
                        ).wait()
                return carry2

            lax.fori_loop(0, (nblk + 3) // 4, bat, 0)
            return carry

        lax.fori_loop(0, 14, chunk, 0)
        plsc.subcore_barrier()

        @pl.when(s < 8)
        def _():
            pltpu.sync_copy(
                acc.at[pl.ds(s * 392, 392)],
                s2_hbm.at[pl.ds(lo + s * 392, 392)],
            )

        plsc.subcore_barrier()


def _sc_edge(xs2, srcp, selp, dstp, intp):
    mesh = plsc.VectorSubcoreMesh(
        core_axis_name="c", subcore_axis_name="s", num_cores=2, num_subcores=16
    )
    f = pl.kernel(
        _edge_body,
        out_type=jax.ShapeDtypeStruct((NCP, C), jnp.float32),
        mesh=mesh,
        compiler_params=pltpu.CompilerParams(needs_layout_passes=False),
        scratch_types=[
            pltpu.VMEM_SHARED((EACC_ROWS, C), jnp.float32),
            pltpu.VMEM((2048,), jnp.int32),
            pltpu.VMEM((16, 128), jnp.int32),
            pltpu.VMEM((2048,), jnp.float32),
            pltpu.VMEM((512, C), jnp.float32),
            pltpu.VMEM((1792,), jnp.float32),
            pltpu.VMEM((1792,), jnp.int32),
            pltpu.VMEM((1792,), jnp.int32),
            pltpu.VMEM((1792,), jnp.int32),
            pltpu.VMEM((64, C), jnp.float32),
            pltpu.SemaphoreType.DMA,
            pltpu.SemaphoreType.DMA,
        ],
    )
    return f(xs2, srcp, selp, dstp, intp)


def _tc_pre_body(xp_ref, w1_ref, wr_ref, u1_ref, ur_ref, st_ref):
    i = pl.program_id(0)
    xb = xp_ref[...]
    u1 = jnp.dot(xb, w1_ref[...], preferred_element_type=jnp.float32)
    ur = jnp.dot(xb, wr_ref[...], preferred_element_type=jnp.float32)
    u1_ref[...] = u1
    ur_ref[...] = ur
    st = jnp.stack(
        [u1.sum(0), (u1 * u1).sum(0), ur.sum(0), (ur * ur).sum(0)]
    )

    @pl.when(i == 0)
    def _():
        st_ref[...] = st

    @pl.when(i > 0)
    def _():
        st_ref[...] = st_ref[...] + st


def _tc_pre(xp, W1, Wr):
    return pl.pallas_call(
        _tc_pre_body,
        grid=(32,),
        in_specs=[
            pl.BlockSpec((784, C), lambda i: (i, 0)),
            pl.BlockSpec((C, C), lambda i: (0, 0)),
            pl.BlockSpec((C, C), lambda i: (0, 0)),
        ],
        out_specs=[
            pl.BlockSpec((784, C), lambda i: (i, 0)),
            pl.BlockSpec((784, C), lambda i: (i, 0)),
            pl.BlockSpec((4, C), lambda i: (0, 0)),
        ],
        out_shape=[
            jax.ShapeDtypeStruct((NCP, C), jnp.float32),
            jax.ShapeDtypeStruct((NCP, C), jnp.float32),
            jax.ShapeDtypeStruct((4, C), jnp.float32),
        ],
    )(xp, W1, Wr)


def _tc_mid_body(u1_ref, st_ref, g1_ref, be1_ref, w2_ref, xs_ref):
    mu = st_ref[0, :] * (1.0 / NC)
    var = st_ref[1, :] * (1.0 / NC) - mu * mu
    inv = lax.rsqrt(var + EPS)
    h = jnp.maximum(
        (u1_ref[...] - mu[None, :]) * (inv * g1_ref[0, :])[None, :]
        + be1_ref[0, :][None, :],
        0.0,
    )
    for s9 in range(9):
        xs_ref[s9] = jnp.dot(h, w2_ref[s9], preferred_element_type=jnp.float32)


def _tc_mid(U1, st, g1, be1, W2):
    return pl.pallas_call(
        _tc_mid_body,
        grid=(32,),
        in_specs=[
            pl.BlockSpec((784, C), lambda i: (i, 0)),
            pl.BlockSpec((4, C), lambda i: (0, 0)),
            pl.BlockSpec((1, C), lambda i: (0, 0)),
            pl.BlockSpec((1, C), lambda i: (0, 0)),
            pl.BlockSpec((9, C, C), lambda i: (0, 0, 0)),
        ],
        out_specs=pl.BlockSpec((9, 784, C), lambda i: (0, i, 0)),
        out_shape=jax.ShapeDtypeStruct((9, NCP, C), jnp.float32),
    )(U1, st, g1.reshape(1, C), be1.reshape(1, C), W2)


def _tc_stats_body(s2_ref, st_ref):
    i = pl.program_id(0)
    xb = s2_ref[...]
    st = jnp.stack([xb.sum(0), (xb * xb).sum(0)])

    @pl.when(i == 0)
    def _():
        st_ref[...] = st

    @pl.when(i > 0)
    def _():
        st_ref[...] = st_ref[...] + st


def _tc_stats(S2):
    return pl.pallas_call(
        _tc_stats_body,
        grid=(32,),
        in_specs=[pl.BlockSpec((784, C), lambda i: (i, 0))],
        out_specs=pl.BlockSpec((2, C), lambda i: (0, 0)),
        out_shape=jax.ShapeDtypeStruct((2, C), jnp.float32),
    )(S2)


def _tc_post_body(s2_ref, st2_ref, g2_ref, be2_ref, w3_ref, b3_ref,
                  ur_ref, str_ref, gr_ref, ber_ref, y_ref):
    mu2 = st2_ref[0, :] * (1.0 / NC)
    var2 = st2_ref[1, :] * (1.0 / NC) - mu2 * mu2
    inv2 = lax.rsqrt(var2 + EPS)
    s2 = s2_ref[...]
    o3 = jnp.maximum(
        (s2 - mu2[None, :]) * (inv2 * g2_ref[0, :])[None, :]
        + be2_ref[0, :][None, :],
        0.0,
    )
    out4 = jnp.dot(o3, w3_ref[...], preferred_element_type=jnp.float32)
    out4 = out4 + b3_ref[0, :][None, :]
    mur = str_ref[2, :] * (1.0 / NC)
    varr = str_ref[3, :] * (1.0 / NC) - mur * mur
    invr = lax.rsqrt(varr + EPS)
    res = (ur_ref[...] - mur[None, :]) * (invr * gr_ref[0, :])[None, :] \
        + ber_ref[0, :][None, :]
    y_ref[...] = jnp.maximum(out4 + res, 0.0)


def _tc_post(S2, st2, g2, be2, W3, b3, Ur, st, gr, ber):
    return pl.pallas_call(
        _tc_post_body,
        grid=(25,),
        in_specs=[
            pl.BlockSpec((1000, C), lambda i: (i, 0)),
            pl.BlockSpec((2, C), lambda i: (0, 0)),
            pl.BlockSpec((1, C), lambda i: (0, 0)),
            pl.BlockSpec((1, C), lambda i: (0, 0)),
            pl.BlockSpec((C, C), lambda i: (0, 0)),
            pl.BlockSpec((1, C), lambda i: (0, 0)),
            pl.BlockSpec((1000, C), lambda i: (i, 0)),
            pl.BlockSpec((4, C), lambda i: (0, 0)),
            pl.BlockSpec((1, C), lambda i: (0, 0)),
            pl.BlockSpec((1, C), lambda i: (0, 0)),
        ],
        out_specs=pl.BlockSpec((1000, C), lambda i: (i, 0)),
        out_shape=jax.ShapeDtypeStruct((NC, C), jnp.float32),
    )(S2, st2, g2.reshape(1, C), be2.reshape(1, C), W3, b3.reshape(1, C),
      Ur, st, gr.reshape(1, C), ber.reshape(1, C))


def kernel(x, cluster, down_edge_index, down_selections, down_interps,
           W1, b1, g1, be1, W2, b2, g2, be2, W3, b3, Wr, br, gr, ber):
    del b1, b2, br

    clp = jnp.concatenate(
        [cluster, jnp.full((NP - N,), NCP, jnp.int32)]
    )
    xp = _sc_segmax(x, clp)

    U1, Ur, st = _tc_pre(xp, W1, Wr)
    xs = _tc_mid(U1, st, g1, be1, W2)

    src = down_edge_index[0]
    dst = down_edge_index[1]
    pad = EP - E
    srcp = jnp.concatenate([src, jnp.zeros((pad,), jnp.int32)])
    dstp = jnp.concatenate([dst, jnp.full((pad,), NCP, jnp.int32)])
    selp = jnp.concatenate([down_selections, jnp.zeros((pad,), jnp.int32)])
    intp = jnp.concatenate([down_interps, jnp.zeros((pad,), jnp.float32)])

    xs2 = xs.reshape(9 * NCP, C)
    S2 = _sc_edge(xs2, srcp, selp, dstp, intp)

    st2 = _tc_stats(S2)
    return _tc_post(S2, st2, g2, be2, W3, b3, Ur, st, gr, ber)
